# baseline (speedup 1.0000x reference)
import functools

import jax
import jax.numpy as jnp
from jax import lax
from jax.experimental import pallas as pl
from jax.experimental.pallas import tpu as pltpu
from jax.experimental.pallas import tpu_sc as plsc

N = 10000
E = 320000
H = 128

N_PAD = 10240
CHUNK = 128
NW = 32
NBUF = 2
EDGE_CHUNKS = -(-E // (NW * CHUNK * NBUF)) * NBUF
E_PAD = NW * CHUNK * EDGE_CHUNKS
BCHUNK = 64
BNBUF = 4
B_EDGE_CHUNKS = E_PAD // (NW * BCHUNK)
ROWS_PER_TILE = N_PAD // 16


def _degree_body(dst_hbm, ones_hbm, zeros_hbm, out_hbm,
                 didx_bufs, ones_v, cnt_sh, isems, ssems):
    c = lax.axis_index("c")
    s = lax.axis_index("s")
    wid = c * 16 + s

    pltpu.sync_copy(ones_hbm, ones_v)
    r0 = pl.multiple_of(s * ROWS_PER_TILE, CHUNK)
    pltpu.sync_copy(zeros_hbm, cnt_sh.at[pl.ds(r0, ROWS_PER_TILE)])
    plsc.subcore_barrier()

    def edge_body(g, _):
        e0 = pl.multiple_of((wid * EDGE_CHUNKS + g * NBUF) * CHUNK, CHUNK)
        ld = [
            pltpu.async_copy(dst_hbm.at[pl.ds(e0 + b * CHUNK, CHUNK)],
                             didx_bufs[b], isems[b])
            for b in range(NBUF)
        ]
        ss = []
        for b in range(NBUF):
            ld[b].wait()
            ss.append(
                pltpu.async_copy(ones_v, cnt_sh.at[didx_bufs[b]], ssems[b],
                                 add=True))
        for d in ss:
            d.wait()
        return 0

    lax.fori_loop(0, EDGE_CHUNKS // NBUF, edge_body, 0)
    plsc.subcore_barrier()

    pltpu.sync_copy(cnt_sh.at[pl.ds(r0, ROWS_PER_TILE)],
                    out_hbm.at[c, pl.ds(r0, ROWS_PER_TILE)])


@functools.lru_cache(maxsize=None)
def _build_sc_degree():
    mesh = plsc.VectorSubcoreMesh(core_axis_name="c", subcore_axis_name="s")
    return pl.kernel(
        _degree_body,
        out_type=jax.ShapeDtypeStruct((2, N_PAD, H), jnp.float32),
        mesh=mesh,
        scratch_types=[
            [pltpu.VMEM((CHUNK,), jnp.int32)] * NBUF,
            pltpu.VMEM((CHUNK, H), jnp.float32),
            pltpu.VMEM_SHARED((N_PAD, H), jnp.float32),
            [pltpu.SemaphoreType.DMA] * NBUF,
            [pltpu.SemaphoreType.DMA] * NBUF,
        ],
    )


def _sc_degree(dst_p):
    ones = jnp.ones((CHUNK, H), jnp.float32)
    zeros = jnp.zeros((ROWS_PER_TILE, H), jnp.float32)
    return _build_sc_degree()(dst_p, ones, zeros)


def _aggregate_body(src_hbm, dst_hbm, hw2_hbm, zeros_hbm, out_hbm,
                    sidx_bufs, didx_bufs, rows_bufs, agg_sh,
                    isems, gsems, ssems):
    c = lax.axis_index("c")
    s = lax.axis_index("s")
    wid = c * 16 + s

    r0 = pl.multiple_of(s * ROWS_PER_TILE, CHUNK)
    pltpu.sync_copy(zeros_hbm, agg_sh.at[pl.ds(r0, ROWS_PER_TILE)])
    plsc.subcore_barrier()

    def edge_body(g, _):
        e0 = pl.multiple_of((wid * B_EDGE_CHUNKS + g * BNBUF) * BCHUNK,
                            BCHUNK)
        ld = [
            pltpu.async_copy(src_hbm.at[pl.ds(e0 + b * BCHUNK, BCHUNK)],
                             sidx_bufs[b], isems[2 * b])
            for b in range(BNBUF)
        ] + [
            pltpu.async_copy(dst_hbm.at[pl.ds(e0 + b * BCHUNK, BCHUNK)],
                             didx_bufs[b], isems[2 * b + 1])
            for b in range(BNBUF)
        ]
        gs = []
        for b in range(BNBUF):
            ld[b].wait()
            gs.append(
                pltpu.async_copy(hw2_hbm.at[sidx_bufs[b]], rows_bufs[b],
                                 gsems[b]))
        ss = []
        for b in range(BNBUF):
            gs[b].wait()
            ld[BNBUF + b].wait()
            ss.append(
                pltpu.async_copy(rows_bufs[b], agg_sh.at[didx_bufs[b]],
                                 ssems[b], add=True))
        for d in ss:
            d.wait()
        return 0

    lax.fori_loop(0, B_EDGE_CHUNKS // BNBUF, edge_body, 0)
    plsc.subcore_barrier()

    pltpu.sync_copy(agg_sh.at[pl.ds(r0, ROWS_PER_TILE)],
                    out_hbm.at[c, pl.ds(r0, ROWS_PER_TILE)])


@functools.lru_cache(maxsize=None)
def _build_sc_aggregate():
    mesh = plsc.VectorSubcoreMesh(core_axis_name="c", subcore_axis_name="s")
    return pl.kernel(
        _aggregate_body,
        out_type=jax.ShapeDtypeStruct((2, N_PAD, H), jnp.float32),
        mesh=mesh,
        scratch_types=[
            [pltpu.VMEM((BCHUNK,), jnp.int32)] * BNBUF,
            [pltpu.VMEM((BCHUNK,), jnp.int32)] * BNBUF,
            [pltpu.VMEM((BCHUNK, H), jnp.float32)] * BNBUF,
            pltpu.VMEM_SHARED((N_PAD, H), jnp.float32),
            [pltpu.SemaphoreType.DMA] * (2 * BNBUF),
            [pltpu.SemaphoreType.DMA] * BNBUF,
            [pltpu.SemaphoreType.DMA] * BNBUF,
        ],
    )


def _sc_aggregate(src_p, dst_p, hw2_p):
    zeros = jnp.zeros((ROWS_PER_TILE, H), jnp.float32)
    return _build_sc_aggregate()(src_p, dst_p, hw2_p, zeros)


BLK = 1000


def _k1_body(cnt_ref, x_ref, wenc_t_ref, benc_ref, wgcn_t_ref,
             hw2_ref, dinv_ref):
    deg = cnt_ref[0, :, 0:1] + cnt_ref[1, :, 0:1] + 1.0
    dinv = lax.rsqrt(deg)
    h = jnp.dot(x_ref[...], wenc_t_ref[...],
                preferred_element_type=jnp.float32) + benc_ref[...]
    hw = jnp.dot(h, wgcn_t_ref[...], preferred_element_type=jnp.float32)
    hw2_ref[...] = hw * dinv
    dinv_ref[...] = dinv


def _run_k1(cnt, x, wenc_t, benc, wgcn_t):
    return pl.pallas_call(
        _k1_body,
        grid=(N // BLK,),
        in_specs=[
            pl.BlockSpec((2, BLK, 1), lambda i: (0, i, 0)),
            pl.BlockSpec((BLK, H), lambda i: (i, 0)),
            pl.BlockSpec((H, H), lambda i: (0, 0)),
            pl.BlockSpec((1, H), lambda i: (0, 0)),
            pl.BlockSpec((H, H), lambda i: (0, 0)),
        ],
        out_specs=[
            pl.BlockSpec((BLK, H), lambda i: (i, 0)),
            pl.BlockSpec((BLK, 1), lambda i: (i, 0)),
        ],
        out_shape=[
            jax.ShapeDtypeStruct((N, H), jnp.float32),
            jax.ShapeDtypeStruct((N, 1), jnp.float32),
        ],
    )(cnt, x, wenc_t, benc, wgcn_t)


def _k2_body(agg_ref, hw2_ref, dinv_ref, bgcn_ref, wih_t_ref, bih_ref,
             gi_ref):
    a = agg_ref[0] + agg_ref[1] + hw2_ref[...]
    gcn = a * dinv_ref[...] + bgcn_ref[...]
    gi_ref[...] = jnp.dot(gcn, wih_t_ref[...],
                          preferred_element_type=jnp.float32) + bih_ref[...]


def _run_k2(agg, hw2, dinv, bgcn, wih_t, bih):
    return pl.pallas_call(
        _k2_body,
        grid=(N // BLK,),
        in_specs=[
            pl.BlockSpec((2, BLK, H), lambda i: (0, i, 0)),
            pl.BlockSpec((BLK, H), lambda i: (i, 0)),
            pl.BlockSpec((BLK, 1), lambda i: (i, 0)),
            pl.BlockSpec((1, H), lambda i: (0, 0)),
            pl.BlockSpec((H, 3 * H), lambda i: (0, 0)),
            pl.BlockSpec((1, 3 * H), lambda i: (0, 0)),
        ],
        out_specs=pl.BlockSpec((BLK, 3 * H), lambda i: (i, 0)),
        out_shape=jax.ShapeDtypeStruct((N, 3 * H), jnp.float32),
    )(agg, hw2, dinv, bgcn, wih_t, bih)


def _k3_body(gi_ref, whh_t_ref, bhhn_ref, wpol_t_ref, bpol_ref,
             out_ref, hcarry_ref, hbuf_ref):
    pid = pl.program_id(0)

    @pl.when(pid == 0)
    def _():
        hcarry_ref[...] = jnp.zeros((8, H), jnp.float32)

    whh_t = whh_t_ref[...]
    bhhn = bhhn_ref[...]

    def step8(j, hprev):
        i0 = pl.multiple_of(j * 8, 8)
        g8 = gi_ref[pl.ds(i0, 8), :]
        rows = []
        h = hprev
        for b in range(8):
            gh = jnp.dot(h.astype(jnp.bfloat16), whh_t,
                         preferred_element_type=jnp.float32)
            g = g8[b:b + 1, :] + gh
            r = jax.nn.sigmoid(g[:, 0:H])
            z = jax.nn.sigmoid(g[:, H:2 * H])
            hn = gh[:, 2 * H:3 * H] + bhhn
            nn_ = jnp.tanh(g8[b:b + 1, 2 * H:3 * H] + hn * r)
            h = nn_ + z * (h - nn_)
            rows.append(h[0:1, :])
        hbuf_ref[pl.ds(i0, 8), :] = jnp.concatenate(rows, axis=0)
        return h

    h0 = hcarry_ref[...]
    hfin = lax.fori_loop(0, BLK // 8, step8, h0)
    hcarry_ref[...] = hfin
    out_ref[...] = jnp.dot(hbuf_ref[...], wpol_t_ref[...],
                           preferred_element_type=jnp.float32) + bpol_ref[...]


def _run_k3(gi, whh_t, bhhn, wpol_t, bpol):
    whh_t = whh_t.astype(jnp.bfloat16)
    return pl.pallas_call(
        _k3_body,
        grid=(N // BLK,),
        in_specs=[
            pl.BlockSpec((BLK, 3 * H), lambda i: (i, 0)),
            pl.BlockSpec((H, 3 * H), lambda i: (0, 0)),
            pl.BlockSpec((1, H), lambda i: (0, 0)),
            pl.BlockSpec((H, 1), lambda i: (0, 0)),
            pl.BlockSpec((1, 1), lambda i: (0, 0)),
        ],
        out_specs=pl.BlockSpec((BLK, 1), lambda i: (i, 0)),
        out_shape=jax.ShapeDtypeStruct((N, 1), jnp.float32),
        scratch_shapes=[
            pltpu.VMEM((8, H), jnp.float32),
            pltpu.VMEM((BLK, H), jnp.float32),
        ],
    )(gi, whh_t, bhhn, wpol_t, bpol)


@jax.jit
def kernel(x, edge_index, W_enc, b_enc, W_gcn, b_gcn, W_ih, W_hh, b_ih, b_hh,
           W_pol, b_pol):
    src = edge_index[0]
    dst = edge_index[1]
    pad = E_PAD - E
    src_p = jnp.concatenate([src, jnp.full((pad,), N, jnp.int32)])
    dst_p = jnp.concatenate([dst, jnp.full((pad,), N, jnp.int32)])

    cnt = _sc_degree(dst_p)

    hw2, dinv = _run_k1(cnt[:, :N, :1], x, W_enc.T, b_enc.reshape(1, H),
                        W_gcn.T)

    hw2_p = jnp.concatenate([hw2, jnp.zeros((N_PAD - N, H), jnp.float32)])
    agg = _sc_aggregate(src_p, dst_p, hw2_p)

    bias3 = b_ih + jnp.concatenate(
        [b_hh[0:2 * H], jnp.zeros((H,), jnp.float32)])
    gi = _run_k2(agg[:, :N, :], hw2, dinv, b_gcn.reshape(1, H), W_ih.T,
                 bias3.reshape(1, 3 * H))

    scores = _run_k3(gi, W_hh.T, b_hh[2 * H:3 * H].reshape(1, H), W_pol.T,
                     b_pol.reshape(1, 1))
    return scores[:, 0]


# speedup vs baseline: 1.0504x; 1.0504x over previous
import functools

import jax
import jax.numpy as jnp
from jax import lax
from jax.experimental import pallas as pl
from jax.experimental.pallas import tpu as pltpu
from jax.experimental.pallas import tpu_sc as plsc

N = 10000
E = 320000
H = 128

N_PAD = 10240
CHUNK = 128
NW = 32
NBUF = 2
EDGE_CHUNKS = -(-E // (NW * CHUNK * NBUF)) * NBUF
E_PAD = NW * CHUNK * EDGE_CHUNKS
BCHUNK = 64
BNBUF = 4
B_TOTAL_PAIR = E_PAD // (16 * BCHUNK)
B_N0 = 252
B_N1 = B_TOTAL_PAIR - B_N0
ROWS_PER_TILE = N_PAD // 16


def _degree_body(dst_hbm, ones_hbm, zeros_hbm, out_hbm,
                 didx_bufs, ones_v, cnt_sh, isems, ssems):
    c = lax.axis_index("c")
    s = lax.axis_index("s")
    wid = c * 16 + s

    pltpu.sync_copy(ones_hbm, ones_v)
    r0 = pl.multiple_of(s * ROWS_PER_TILE, CHUNK)
    pltpu.sync_copy(zeros_hbm, cnt_sh.at[pl.ds(r0, ROWS_PER_TILE)])
    plsc.subcore_barrier()

    def edge_body(g, _):
        e0 = pl.multiple_of((wid * EDGE_CHUNKS + g * NBUF) * CHUNK, CHUNK)
        ld = [
            pltpu.async_copy(dst_hbm.at[pl.ds(e0 + b * CHUNK, CHUNK)],
                             didx_bufs[b], isems[b])
            for b in range(NBUF)
        ]
        ss = []
        for b in range(NBUF):
            ld[b].wait()
            ss.append(
                pltpu.async_copy(ones_v, cnt_sh.at[didx_bufs[b]], ssems[b],
                                 add=True))
        for d in ss:
            d.wait()
        return 0

    lax.fori_loop(0, EDGE_CHUNKS // NBUF, edge_body, 0)
    plsc.subcore_barrier()

    pltpu.sync_copy(cnt_sh.at[pl.ds(r0, ROWS_PER_TILE)],
                    out_hbm.at[c, pl.ds(r0, ROWS_PER_TILE)])


@functools.lru_cache(maxsize=None)
def _build_sc_degree():
    mesh = plsc.VectorSubcoreMesh(core_axis_name="c", subcore_axis_name="s")
    return pl.kernel(
        _degree_body,
        out_type=jax.ShapeDtypeStruct((2, N_PAD, H), jnp.float32),
        mesh=mesh,
        scratch_types=[
            [pltpu.VMEM((CHUNK,), jnp.int32)] * NBUF,
            pltpu.VMEM((CHUNK, H), jnp.float32),
            pltpu.VMEM_SHARED((N_PAD, H), jnp.float32),
            [pltpu.SemaphoreType.DMA] * NBUF,
            [pltpu.SemaphoreType.DMA] * NBUF,
        ],
    )


def _sc_degree(dst_p):
    ones = jnp.ones((CHUNK, H), jnp.float32)
    zeros = jnp.zeros((ROWS_PER_TILE, H), jnp.float32)
    return _build_sc_degree()(dst_p, ones, zeros)


def _aggregate_body(src_hbm, dst_hbm, hw2_hbm, zeros_hbm, out_hbm,
                    sidx_bufs, didx_bufs, rows_bufs, agg_sh,
                    isems, gsems, ssems):
    c = lax.axis_index("c")
    s = lax.axis_index("s")
    wid = c * 16 + s

    r0 = pl.multiple_of(s * ROWS_PER_TILE, CHUNK)
    pltpu.sync_copy(zeros_hbm, agg_sh.at[pl.ds(r0, ROWS_PER_TILE)])
    plsc.subcore_barrier()

    base_ck = jnp.where(c == 0, s * B_N0, 16 * B_N0 + s * B_N1)
    n_groups = jnp.where(c == 0, B_N0 // BNBUF, B_N1 // BNBUF)

    def edge_body(g, _):
        e0 = pl.multiple_of((base_ck + g * BNBUF) * BCHUNK, BCHUNK)
        ld = [
            pltpu.async_copy(src_hbm.at[pl.ds(e0 + b * BCHUNK, BCHUNK)],
                             sidx_bufs[b], isems[2 * b])
            for b in range(BNBUF)
        ] + [
            pltpu.async_copy(dst_hbm.at[pl.ds(e0 + b * BCHUNK, BCHUNK)],
                             didx_bufs[b], isems[2 * b + 1])
            for b in range(BNBUF)
        ]
        gs = []
        for b in range(BNBUF):
            ld[b].wait()
            gs.append(
                pltpu.async_copy(hw2_hbm.at[sidx_bufs[b]], rows_bufs[b],
                                 gsems[b]))
        ss = []
        for b in range(BNBUF):
            gs[b].wait()
            ld[BNBUF + b].wait()
            ss.append(
                pltpu.async_copy(rows_bufs[b], agg_sh.at[didx_bufs[b]],
                                 ssems[b], add=True))
        for d in ss:
            d.wait()
        return 0

    lax.fori_loop(0, n_groups, edge_body, 0)
    plsc.subcore_barrier()

    pltpu.sync_copy(agg_sh.at[pl.ds(r0, ROWS_PER_TILE)],
                    out_hbm.at[c, pl.ds(r0, ROWS_PER_TILE)])


@functools.lru_cache(maxsize=None)
def _build_sc_aggregate():
    mesh = plsc.VectorSubcoreMesh(core_axis_name="c", subcore_axis_name="s")
    return pl.kernel(
        _aggregate_body,
        out_type=jax.ShapeDtypeStruct((2, N_PAD, H), jnp.float32),
        mesh=mesh,
        scratch_types=[
            [pltpu.VMEM((BCHUNK,), jnp.int32)] * BNBUF,
            [pltpu.VMEM((BCHUNK,), jnp.int32)] * BNBUF,
            [pltpu.VMEM((BCHUNK, H), jnp.float32)] * BNBUF,
            pltpu.VMEM_SHARED((N_PAD, H), jnp.float32),
            [pltpu.SemaphoreType.DMA] * (2 * BNBUF),
            [pltpu.SemaphoreType.DMA] * BNBUF,
            [pltpu.SemaphoreType.DMA] * BNBUF,
        ],
    )


def _sc_aggregate(src_p, dst_p, hw2_p):
    zeros = jnp.zeros((ROWS_PER_TILE, H), jnp.float32)
    return _build_sc_aggregate()(src_p, dst_p, hw2_p, zeros)


BLK = 1000


def _k1_body(cnt_ref, x_ref, wenc_t_ref, benc_ref, wgcn_t_ref,
             hw2_ref, dinv_ref):
    deg = cnt_ref[0, :, 0:1] + cnt_ref[1, :, 0:1] + 1.0
    dinv = lax.rsqrt(deg)
    h = jnp.dot(x_ref[...], wenc_t_ref[...],
                preferred_element_type=jnp.float32) + benc_ref[...]
    hw = jnp.dot(h, wgcn_t_ref[...], preferred_element_type=jnp.float32)
    hw2_ref[...] = hw * dinv
    dinv_ref[...] = dinv


def _run_k1(cnt, x, wenc_t, benc, wgcn_t):
    return pl.pallas_call(
        _k1_body,
        grid=(N // BLK,),
        in_specs=[
            pl.BlockSpec((2, BLK, 1), lambda i: (0, i, 0)),
            pl.BlockSpec((BLK, H), lambda i: (i, 0)),
            pl.BlockSpec((H, H), lambda i: (0, 0)),
            pl.BlockSpec((1, H), lambda i: (0, 0)),
            pl.BlockSpec((H, H), lambda i: (0, 0)),
        ],
        out_specs=[
            pl.BlockSpec((BLK, H), lambda i: (i, 0)),
            pl.BlockSpec((BLK, 1), lambda i: (i, 0)),
        ],
        out_shape=[
            jax.ShapeDtypeStruct((N, H), jnp.float32),
            jax.ShapeDtypeStruct((N, 1), jnp.float32),
        ],
    )(cnt, x, wenc_t, benc, wgcn_t)


def _k2_body(agg_ref, hw2_ref, dinv_ref, bgcn_ref, wih_t_ref, bih_ref,
             gi_ref):
    a = agg_ref[0] + agg_ref[1] + hw2_ref[...]
    gcn = a * dinv_ref[...] + bgcn_ref[...]
    gi_ref[...] = jnp.dot(gcn, wih_t_ref[...],
                          preferred_element_type=jnp.float32) + bih_ref[...]


def _run_k2(agg, hw2, dinv, bgcn, wih_t, bih):
    return pl.pallas_call(
        _k2_body,
        grid=(N // BLK,),
        in_specs=[
            pl.BlockSpec((2, BLK, H), lambda i: (0, i, 0)),
            pl.BlockSpec((BLK, H), lambda i: (i, 0)),
            pl.BlockSpec((BLK, 1), lambda i: (i, 0)),
            pl.BlockSpec((1, H), lambda i: (0, 0)),
            pl.BlockSpec((H, 3 * H), lambda i: (0, 0)),
            pl.BlockSpec((1, 3 * H), lambda i: (0, 0)),
        ],
        out_specs=pl.BlockSpec((BLK, 3 * H), lambda i: (i, 0)),
        out_shape=jax.ShapeDtypeStruct((N, 3 * H), jnp.float32),
    )(agg, hw2, dinv, bgcn, wih_t, bih)


def _k3_body(gi_ref, whh_t_ref, bhhn_ref, wpol_t_ref, bpol_ref,
             out_ref, hcarry_ref, hbuf_ref):
    pid = pl.program_id(0)

    @pl.when(pid == 0)
    def _():
        hcarry_ref[...] = jnp.zeros((8, H), jnp.float32)

    whh_t = whh_t_ref[...]
    bhhn = bhhn_ref[...]

    def step8(j, hprev):
        i0 = pl.multiple_of(j * 8, 8)
        g8 = gi_ref[pl.ds(i0, 8), :]
        rows = []
        h = hprev
        for b in range(8):
            gh = jnp.dot(h, whh_t, preferred_element_type=jnp.float32)
            g = g8[b:b + 1, :] + gh
            r = jax.nn.sigmoid(g[:, 0:H])
            z = jax.nn.sigmoid(g[:, H:2 * H])
            hn = gh[:, 2 * H:3 * H] + bhhn
            nn_ = jnp.tanh(g8[b:b + 1, 2 * H:3 * H] + hn * r)
            h = nn_ + z * (h - nn_)
            rows.append(h[0:1, :])
        hbuf_ref[pl.ds(i0, 8), :] = jnp.concatenate(rows, axis=0)
        return h

    h0 = hcarry_ref[...]
    hfin = lax.fori_loop(0, BLK // 8, step8, h0)
    hcarry_ref[...] = hfin
    out_ref[...] = jnp.dot(hbuf_ref[...], wpol_t_ref[...],
                           preferred_element_type=jnp.float32) + bpol_ref[...]


def _run_k3(gi, whh_t, bhhn, wpol_t, bpol):
    return pl.pallas_call(
        _k3_body,
        grid=(N // BLK,),
        in_specs=[
            pl.BlockSpec((BLK, 3 * H), lambda i: (i, 0)),
            pl.BlockSpec((H, 3 * H), lambda i: (0, 0)),
            pl.BlockSpec((1, H), lambda i: (0, 0)),
            pl.BlockSpec((H, 1), lambda i: (0, 0)),
            pl.BlockSpec((1, 1), lambda i: (0, 0)),
        ],
        out_specs=pl.BlockSpec((BLK, 1), lambda i: (i, 0)),
        out_shape=jax.ShapeDtypeStruct((N, 1), jnp.float32),
        scratch_shapes=[
            pltpu.VMEM((8, H), jnp.float32),
            pltpu.VMEM((BLK, H), jnp.float32),
        ],
    )(gi, whh_t, bhhn, wpol_t, bpol)


@jax.jit
def kernel(x, edge_index, W_enc, b_enc, W_gcn, b_gcn, W_ih, W_hh, b_ih, b_hh,
           W_pol, b_pol):
    src = edge_index[0]
    dst = edge_index[1]
    pad = E_PAD - E
    src_p = jnp.concatenate([src, jnp.full((pad,), N, jnp.int32)])
    dst_p = jnp.concatenate([dst, jnp.full((pad,), N, jnp.int32)])

    cnt = _sc_degree(dst_p)

    hw2, dinv = _run_k1(cnt[:, :N, :1], x, W_enc.T, b_enc.reshape(1, H),
                        W_gcn.T)

    hw2_p = jnp.concatenate([hw2, jnp.zeros((N_PAD - N, H), jnp.float32)])
    agg = _sc_aggregate(src_p, dst_p, hw2_p)

    bias3 = b_ih + jnp.concatenate(
        [b_hh[0:2 * H], jnp.zeros((H,), jnp.float32)])
    gi = _run_k2(agg[:, :N, :], hw2, dinv, b_gcn.reshape(1, H), W_ih.T,
                 bias3.reshape(1, 3 * H))

    scores = _run_k3(gi, W_hh.T, b_hh[2 * H:3 * H].reshape(1, H), W_pol.T,
                     b_pol.reshape(1, 1))
    return scores[:, 0]
